# Initial kernel scaffold; baseline (speedup 1.0000x reference)
import jax
import jax.numpy as jnp
from jax.experimental import pallas as pl


def kernel(boxes, scores):
    raise NotImplementedError("write your pallas kernel here")


# speedup vs baseline: 16.3875x; 16.3875x over previous
import functools

import jax
import jax.numpy as jnp
from jax.experimental import pallas as pl

_N = 20000
_K = 6000
_NOUT = 300
_IOU = 0.7
_SCALE = 1000.0
_ROWS = 160
_LANES = 128
_P = _ROWS * _LANES
_NEG = -1e9


def _nms_body(c0_ref, c1_ref, c2_ref, c3_ref, s_ref, out_ref):
    f32 = jnp.float32
    i32 = jnp.int32
    imin = jnp.int32(-2147483648)

    row_i = jax.lax.broadcasted_iota(i32, (_ROWS, _LANES), 0)
    lane_i = jax.lax.broadcasted_iota(i32, (_ROWS, _LANES), 1)
    flat_i = row_i * _LANES + lane_i
    valid = flat_i < _N

    b0 = c0_ref[:] * _SCALE
    b1 = c1_ref[:] * _SCALE
    b2 = c2_ref[:] * _SCALE
    b3 = c3_ref[:] * _SCALE
    x1 = jnp.minimum(b0, b2)
    x2 = jnp.maximum(b0, b2)
    y1 = jnp.minimum(b1, b3)
    y2 = jnp.maximum(b1, b3)
    scores = s_ref[:]

    bits = jax.lax.bitcast_convert_type(scores, i32)
    akey = bits ^ (jax.lax.shift_right_arithmetic(bits, 31) & jnp.int32(0x7FFFFFFF))
    akey = jnp.where(valid, akey, imin)
    inv = _P - flat_i

    Tf = imin
    Ti = jnp.int32(0)
    for b in range(31, -1, -1):
        if b == 31:
            trial = Tf ^ imin
        else:
            trial = Tf | jnp.int32(1 << b)
        cnt = jnp.sum((akey >= trial).astype(i32))
        Tf = jnp.where(cnt >= _K, trial, Tf)
    for b in range(14, -1, -1):
        trial = Ti | jnp.int32(1 << b)
        cond = (akey > Tf) | ((akey == Tf) & (inv >= trial))
        cnt = jnp.sum(cond.astype(i32))
        Ti = jnp.where(cnt >= _K, trial, Ti)
    in_set = (akey > Tf) | ((akey == Tf) & (inv >= Ti))

    s0 = jnp.where(in_set, scores, f32(_NEG))
    areas = (x2 - x1) * (y2 - y1)
    neg_inf = f32(-jnp.inf)

    def step(i, carry):
        s, dx1, dy1, dx2, dy2, ds = carry
        m = jnp.max(s)
        idx = jnp.min(jnp.where(s == m, flat_i, _P))
        mask2 = flat_i == idx
        sx1 = jnp.max(jnp.where(mask2, x1, neg_inf))
        sy1 = jnp.max(jnp.where(mask2, y1, neg_inf))
        sx2 = jnp.max(jnp.where(mask2, x2, neg_inf))
        sy2 = jnp.max(jnp.where(mask2, y2, neg_inf))
        ssc = jnp.max(jnp.where(mask2, scores, neg_inf))

        is_deg = m == f32(_NEG)
        dx1 = jnp.where(i == 0, sx1, dx1)
        dy1 = jnp.where(i == 0, sy1, dy1)
        dx2 = jnp.where(i == 0, sx2, dx2)
        dy2 = jnp.where(i == 0, sy2, dy2)
        ds = jnp.where(i == 0, ssc, ds)
        sx1 = jnp.where(is_deg, dx1, sx1)
        sy1 = jnp.where(is_deg, dy1, sy1)
        sx2 = jnp.where(is_deg, dx2, sx2)
        sy2 = jnp.where(is_deg, dy2, sy2)
        ssc = jnp.where(is_deg, ds, ssc)

        xx1 = jnp.maximum(sx1, x1)
        yy1 = jnp.maximum(sy1, y1)
        xx2 = jnp.minimum(sx2, x2)
        yy2 = jnp.minimum(sy2, y2)
        w = jnp.maximum(xx2 - xx1, f32(0.0))
        h = jnp.maximum(yy2 - yy1, f32(0.0))
        inter = w * h
        sel_area = (sx2 - sx1) * (sy2 - sy1)
        iou = inter / (areas + sel_area - inter + f32(1e-9))
        s = jnp.where((iou > f32(_IOU)) | mask2, f32(_NEG), s)

        lane = jax.lax.broadcasted_iota(i32, (1, _LANES), 1)
        row = (
            jnp.where(lane == 0, sx1, f32(0.0))
            + jnp.where(lane == 1, sy1, f32(0.0))
            + jnp.where(lane == 2, sx2, f32(0.0))
            + jnp.where(lane == 3, sy2, f32(0.0))
            + jnp.where(lane == 4, ssc, f32(0.0))
        )
        out_ref[pl.ds(i, 1), :] = row
        return (s, dx1, dy1, dx2, dy2, ds)

    zero = f32(0.0)
    jax.lax.fori_loop(0, _NOUT, step, (s0, zero, zero, zero, zero, zero))


@jax.jit
def kernel(boxes, scores):
    pad = _P - _N
    comps = [
        jnp.pad(boxes[:, i], (0, pad)).reshape(_ROWS, _LANES) for i in range(4)
    ]
    s = jnp.pad(scores, (0, pad)).reshape(_ROWS, _LANES)
    out = pl.pallas_call(
        _nms_body,
        out_shape=jax.ShapeDtypeStruct((_NOUT, _LANES), jnp.float32),
    )(*comps, s)
    return out[:, :5]
